# baseline (speedup 1.0000x reference)
import functools

import jax
import jax.numpy as jnp
from jax import lax
from jax.experimental import pallas as pl
from jax.experimental.pallas import tpu as pltpu
from jax.experimental.pallas import tpu_sc as plsc


def _sc_gather(table, idx_flat):
    info = plsc.get_sparse_core_info()
    nc, ns = info.num_cores, info.num_subcores
    nw = nc * ns
    B = idx_flat.shape[0]
    D = table.shape[1]
    b_per_w = B // nw
    CMAX = 96
    sizes, offs, o = [], [], 0
    while o < b_per_w:
        c = min(CMAX, b_per_w - o)
        sizes.append(c)
        offs.append(o)
        o += c
    mesh = plsc.VectorSubcoreMesh(core_axis_name="c", subcore_axis_name="s")

    @functools.partial(
        pl.kernel,
        mesh=mesh,
        out_type=jax.ShapeDtypeStruct((B, D), jnp.float32),
        scratch_types=[
            pltpu.VMEM((b_per_w,), jnp.int32),
            pltpu.VMEM((CMAX, D), jnp.float32),
            pltpu.SemaphoreType.DMA,
        ],
    )
    def gather_kernel(table_hbm, idx_hbm, out_hbm, idx_v, buf, sem):
        wid = lax.axis_index("s") * nc + lax.axis_index("c")
        base = wid * b_per_w
        pltpu.sync_copy(idx_hbm.at[pl.ds(base, b_per_w)], idx_v)
        for off, c in zip(offs, sizes):
            pltpu.async_copy(
                table_hbm.at[idx_v.at[pl.ds(off, c)]],
                buf.at[pl.ds(0, c)],
                sem,
            ).wait()
            pltpu.sync_copy(buf.at[pl.ds(0, c)], out_hbm.at[pl.ds(base + off, c)])

    return gather_kernel(table, idx_flat)


def _ln_body(w_ref, pos_ref, seg_ref, type_ref, g_ref, b_ref, o_ref):
    x = w_ref[...]
    s = seg_ref[...]
    t0 = type_ref[0, :][None, :]
    t1 = type_ref[1, :][None, :]
    x = x + pos_ref[...] + t0 + s * (t1 - t0)
    mean = jnp.mean(x, axis=1, keepdims=True)
    xc = x - mean
    var = jnp.mean(xc * xc, axis=1, keepdims=True)
    inv = lax.rsqrt(var + 1e-5)
    o_ref[...] = xc * inv * g_ref[...] + b_ref[...]


def _tc_ln(w_e, seg_f, pos_emb, type_emb, gamma, beta, seq):
    B, D = w_e.shape
    R = 1024
    seq_blocks = seq // R
    batch = B // seq
    grid = (seq_blocks, batch)
    return pl.pallas_call(
        _ln_body,
        grid=grid,
        in_specs=[
            pl.BlockSpec((R, D), lambda i, j: (j * seq_blocks + i, 0)),
            pl.BlockSpec((R, D), lambda i, j: (i, 0)),
            pl.BlockSpec((R, 1), lambda i, j: (j * seq_blocks + i, 0)),
            pl.BlockSpec((2, D), lambda i, j: (0, 0)),
            pl.BlockSpec((1, D), lambda i, j: (0, 0)),
            pl.BlockSpec((1, D), lambda i, j: (0, 0)),
        ],
        out_specs=pl.BlockSpec((R, D), lambda i, j: (j * seq_blocks + i, 0)),
        out_shape=jax.ShapeDtypeStruct((B, D), jnp.float32),
    )(w_e, pos_emb, seg_f, type_emb, gamma.reshape(1, D), beta.reshape(1, D))


def _ln_body_acc(prev_ref, w_ref, pos_ref, seg_ref, type_ref, g_ref, b_ref, o_ref):
    del prev_ref
    _ln_body(w_ref, pos_ref, seg_ref, type_ref, g_ref, b_ref, o_ref)


def _tc_ln_half(prev, w_e, seg_f, pos_emb, type_emb, gamma, beta, seq, B_total, b0):
    Bh, D = w_e.shape
    R = 1024
    seq_blocks = seq // R
    batch = Bh // seq
    grid = (seq_blocks, batch)
    return pl.pallas_call(
        _ln_body_acc,
        grid=grid,
        in_specs=[
            pl.BlockSpec(memory_space=pl.ANY),
            pl.BlockSpec((R, D), lambda i, j: (j * seq_blocks + i, 0)),
            pl.BlockSpec((R, D), lambda i, j: (i, 0)),
            pl.BlockSpec((R, 1), lambda i, j: (j * seq_blocks + i, 0)),
            pl.BlockSpec((2, D), lambda i, j: (0, 0)),
            pl.BlockSpec((1, D), lambda i, j: (0, 0)),
            pl.BlockSpec((1, D), lambda i, j: (0, 0)),
        ],
        out_specs=pl.BlockSpec(
            (R, D), lambda i, j, b0=b0: ((b0 + j) * seq_blocks + i, 0)
        ),
        out_shape=jax.ShapeDtypeStruct((B_total, D), jnp.float32),
        input_output_aliases={0: 0},
    )(prev, w_e, pos_emb, seg_f, type_emb, gamma.reshape(1, D), beta.reshape(1, D))


def kernel(input_ids, segment_ids, word_emb, pos_emb, type_emb, ln_gamma, ln_beta):
    batch, seq = input_ids.shape
    D = word_emb.shape[1]
    B = batch * seq
    idx_flat = input_ids.reshape(-1).astype(jnp.int32)
    seg_f = segment_ids.reshape(-1, 1).astype(jnp.float32)
    half = B // 2
    hb = batch // 2
    w_a = _sc_gather(word_emb, idx_flat[:half])
    w_b = _sc_gather(word_emb, idx_flat[half:])
    acc = jnp.zeros((B, D), jnp.float32)
    acc = _tc_ln_half(acc, w_a, seg_f[:half], pos_emb, type_emb,
                      ln_gamma, ln_beta, seq, B, 0)
    acc = _tc_ln_half(acc, w_b, seg_f[half:], pos_emb, type_emb,
                      ln_gamma, ln_beta, seq, B, hb)
    return acc.reshape(batch, seq, D)


# speedup vs baseline: 1.2070x; 1.2070x over previous
import functools

import jax
import jax.numpy as jnp
from jax import lax
from jax.experimental import pallas as pl
from jax.experimental.pallas import tpu as pltpu
from jax.experimental.pallas import tpu_sc as plsc


def _sc_gather(table, idx_flat):
    info = plsc.get_sparse_core_info()
    nc, ns = info.num_cores, info.num_subcores
    nw = nc * ns
    B = idx_flat.shape[0]
    D = table.shape[1]
    b_per_w = B // nw
    CMAX = 96
    sizes, offs, o = [], [], 0
    while o < b_per_w:
        c = min(CMAX, b_per_w - o)
        sizes.append(c)
        offs.append(o)
        o += c
    mesh = plsc.VectorSubcoreMesh(core_axis_name="c", subcore_axis_name="s")

    @functools.partial(
        pl.kernel,
        mesh=mesh,
        out_type=jax.ShapeDtypeStruct((B, D), jnp.float32),
        scratch_types=[
            pltpu.VMEM((b_per_w,), jnp.int32),
            pltpu.VMEM((CMAX, D), jnp.float32),
            pltpu.SemaphoreType.DMA,
        ],
    )
    def gather_kernel(table_hbm, idx_hbm, out_hbm, idx_v, buf, sem):
        wid = lax.axis_index("s") * nc + lax.axis_index("c")
        base = wid * b_per_w
        pltpu.sync_copy(idx_hbm.at[pl.ds(base, b_per_w)], idx_v)
        for off, c in zip(offs, sizes):
            pltpu.async_copy(
                table_hbm.at[idx_v.at[pl.ds(off, c)]],
                buf.at[pl.ds(0, c)],
                sem,
            ).wait()
            pltpu.sync_copy(buf.at[pl.ds(0, c)], out_hbm.at[pl.ds(base + off, c)])

    return gather_kernel(table, idx_flat)


def _ln_body(w_ref, pos_ref, seg_ref, type_ref, g_ref, b_ref, o_ref):
    x = w_ref[...]
    s = seg_ref[...]
    t0 = type_ref[0, :][None, :]
    t1 = type_ref[1, :][None, :]
    x = x + pos_ref[...] + t0 + s * (t1 - t0)
    mean = jnp.mean(x, axis=1, keepdims=True)
    xc = x - mean
    var = jnp.mean(xc * xc, axis=1, keepdims=True)
    inv = lax.rsqrt(var + 1e-5)
    o_ref[...] = xc * inv * g_ref[...] + b_ref[...]


def _tc_ln(w_e, seg_f, pos_emb, type_emb, gamma, beta, seq):
    B, D = w_e.shape
    R = 1024
    seq_blocks = seq // R
    batch = B // seq
    grid = (seq_blocks, batch)
    return pl.pallas_call(
        _ln_body,
        grid=grid,
        in_specs=[
            pl.BlockSpec((R, D), lambda i, j: (j * seq_blocks + i, 0)),
            pl.BlockSpec((R, D), lambda i, j: (i, 0)),
            pl.BlockSpec((R, 1), lambda i, j: (j * seq_blocks + i, 0)),
            pl.BlockSpec((2, D), lambda i, j: (0, 0)),
            pl.BlockSpec((1, D), lambda i, j: (0, 0)),
            pl.BlockSpec((1, D), lambda i, j: (0, 0)),
        ],
        out_specs=pl.BlockSpec((R, D), lambda i, j: (j * seq_blocks + i, 0)),
        out_shape=jax.ShapeDtypeStruct((B, D), jnp.float32),
    )(w_e, pos_emb, seg_f, type_emb, gamma.reshape(1, D), beta.reshape(1, D))


def kernel(input_ids, segment_ids, word_emb, pos_emb, type_emb, ln_gamma, ln_beta):
    batch, seq = input_ids.shape
    D = word_emb.shape[1]
    idx_flat = input_ids.reshape(-1).astype(jnp.int32)
    seg_f = segment_ids.reshape(-1, 1).astype(jnp.float32)
    w_e = _sc_gather(word_emb, idx_flat)
    out = _tc_ln(w_e, seg_f, pos_emb, type_emb, ln_gamma, ln_beta, seq)
    return out.reshape(batch, seq, D)


# speedup vs baseline: 1.2085x; 1.0012x over previous
import functools

import jax
import jax.numpy as jnp
from jax import lax
from jax.experimental import pallas as pl
from jax.experimental.pallas import tpu as pltpu
from jax.experimental.pallas import tpu_sc as plsc


def _sc_gather(table, idx_flat):
    info = plsc.get_sparse_core_info()
    nc, ns = info.num_cores, info.num_subcores
    nw = nc * ns
    B = idx_flat.shape[0]
    D = table.shape[1]
    b_per_w = B // nw
    CMAX = 112
    sizes, offs, o = [], [], 0
    while o < b_per_w:
        c = min(CMAX, b_per_w - o)
        sizes.append(c)
        offs.append(o)
        o += c
    mesh = plsc.VectorSubcoreMesh(core_axis_name="c", subcore_axis_name="s")

    @functools.partial(
        pl.kernel,
        mesh=mesh,
        out_type=jax.ShapeDtypeStruct((B, D), jnp.float32),
        scratch_types=[
            pltpu.VMEM((b_per_w,), jnp.int32),
            pltpu.VMEM((CMAX, D), jnp.float32),
            pltpu.SemaphoreType.DMA,
        ],
    )
    def gather_kernel(table_hbm, idx_hbm, out_hbm, idx_v, buf, sem):
        wid = lax.axis_index("s") * nc + lax.axis_index("c")
        base = wid * b_per_w
        pltpu.sync_copy(idx_hbm.at[pl.ds(base, b_per_w)], idx_v)
        for off, c in zip(offs, sizes):
            pltpu.async_copy(
                table_hbm.at[idx_v.at[pl.ds(off, c)]],
                buf.at[pl.ds(0, c)],
                sem,
            ).wait()
            pltpu.sync_copy(buf.at[pl.ds(0, c)], out_hbm.at[pl.ds(base + off, c)])

    return gather_kernel(table, idx_flat)


def _ln_body(w_ref, pos_ref, seg_ref, type_ref, g_ref, b_ref, o_ref):
    x = w_ref[...]
    s = seg_ref[...].astype(jnp.float32)
    t0 = type_ref[0, :][None, :]
    t1 = type_ref[1, :][None, :]
    x = x + pos_ref[...] + t0 + s * (t1 - t0)
    mean = jnp.mean(x, axis=1, keepdims=True)
    xc = x - mean
    var = jnp.mean(xc * xc, axis=1, keepdims=True)
    inv = lax.rsqrt(var + 1e-5)
    o_ref[...] = xc * inv * g_ref[...] + b_ref[...]


def _tc_ln(w_e, seg_f, pos_emb, type_emb, gamma, beta, seq):
    B, D = w_e.shape
    R = 1024
    seq_blocks = seq // R
    batch = B // seq
    grid = (seq_blocks, batch)
    return pl.pallas_call(
        _ln_body,
        grid=grid,
        in_specs=[
            pl.BlockSpec((R, D), lambda i, j: (j * seq_blocks + i, 0)),
            pl.BlockSpec((R, D), lambda i, j: (i, 0)),
            pl.BlockSpec((R, 1), lambda i, j: (j * seq_blocks + i, 0)),
            pl.BlockSpec((2, D), lambda i, j: (0, 0)),
            pl.BlockSpec((1, D), lambda i, j: (0, 0)),
            pl.BlockSpec((1, D), lambda i, j: (0, 0)),
        ],
        out_specs=pl.BlockSpec((R, D), lambda i, j: (j * seq_blocks + i, 0)),
        out_shape=jax.ShapeDtypeStruct((B, D), jnp.float32),
    )(w_e, pos_emb, seg_f, type_emb, gamma.reshape(1, D), beta.reshape(1, D))


def kernel(input_ids, segment_ids, word_emb, pos_emb, type_emb, ln_gamma, ln_beta):
    batch, seq = input_ids.shape
    D = word_emb.shape[1]
    idx_flat = input_ids.reshape(-1).astype(jnp.int32)
    seg_f = segment_ids.reshape(-1, 1)
    w_e = _sc_gather(word_emb, idx_flat)
    out = _tc_ln(w_e, seg_f, pos_emb, type_emb, ln_gamma, ln_beta, seq)
    return out.reshape(batch, seq, D)
